# Initial kernel scaffold; baseline (speedup 1.0000x reference)
import functools

import jax
import jax.numpy as jnp
from jax import lax
from jax.experimental import pallas as pl
from jax.experimental.pallas import tpu as pltpu
from jax.experimental.pallas import tpu_sc as plsc

_NW = 32


def _mv_body(x_ref, w_ref, b_ref, out_ref):
    out_ref[...] = (
        jnp.sum(x_ref[...] * w_ref[...], axis=1, keepdims=True) + b_ref[0, 0]
    )


def _matvec(X, Wt, b2):
    N, D = X.shape
    BLK = 4096
    return pl.pallas_call(
        _mv_body,
        grid=(N // BLK,),
        in_specs=[
            pl.BlockSpec((BLK, D), lambda i: (i, 0)),
            pl.BlockSpec((1, D), lambda i: (0, 0)),
            pl.BlockSpec(memory_space=pltpu.SMEM),
        ],
        out_specs=pl.BlockSpec((BLK, 1), lambda i: (i, 0)),
        out_shape=jax.ShapeDtypeStruct((N, 1), jnp.float32),
    )(X, Wt, b2)


def _sc_gather(table, bags3):
    nw, R, C = bags3.shape
    mesh = plsc.VectorSubcoreMesh(core_axis_name="c", subcore_axis_name="s")

    @functools.partial(
        pl.kernel,
        out_type=jax.ShapeDtypeStruct((nw, R, C), jnp.float32),
        mesh=mesh,
        scratch_types=[
            pltpu.VMEM((R, C), jnp.int32),
            pltpu.VMEM((R, C), jnp.float32),
            pltpu.SemaphoreType.DMA,
        ],
    )
    def k(table_hbm, bags_hbm, out_hbm, idx_v, vals_v, sem):
        wid = lax.axis_index("s") * 2 + lax.axis_index("c")
        pltpu.sync_copy(bags_hbm.at[wid], idx_v)
        pltpu.async_copy(table_hbm.at[idx_v], vals_v, sem).wait()
        pltpu.sync_copy(vals_v, out_hbm.at[wid])

    return k(table, bags3)


def _fin_body(g_ref, m_ref, out_ref):
    g = g_ref[...]
    msk = m_ref[...]
    g = jnp.where(msk != 0.0, -jnp.inf, g)
    m = jnp.max(g, axis=1, keepdims=True)
    sp = jnp.maximum(m, 0.0) + jnp.log(1.0 + jnp.exp(-jnp.abs(m)))
    out_ref[...] = jnp.concatenate([-sp, m - sp], axis=1)


def _finalize(g, maskf):
    B, L = g.shape
    BLK = 512
    return pl.pallas_call(
        _fin_body,
        grid=(B // BLK,),
        in_specs=[
            pl.BlockSpec((BLK, L), lambda i: (i, 0)),
            pl.BlockSpec((BLK, L), lambda i: (i, 0)),
        ],
        out_specs=pl.BlockSpec((BLK, 2), lambda i: (i, 0)),
        out_shape=jax.ShapeDtypeStruct((B, 2), jnp.float32),
    )(g, maskf)


def kernel(X, bags, padding_mask, W, b):
    N, D = X.shape
    B, L = bags.shape
    logits = _matvec(X, W.reshape(1, D), b.reshape(1, 1))
    table = logits.reshape(N)
    bags3 = bags.astype(jnp.int32).reshape(_NW, (B * L) // _NW // 128, 128)
    gathered = _sc_gather(table, bags3)
    return _finalize(gathered.reshape(B, L), padding_mask.astype(jnp.float32))


# speedup vs baseline: 24.7233x; 24.7233x over previous
import functools

import jax
import jax.numpy as jnp
from jax import lax
from jax.experimental import pallas as pl
from jax.experimental.pallas import tpu as pltpu
from jax.experimental.pallas import tpu_sc as plsc

_NW = 32


def _mv_body(x_ref, w_ref, b_ref, out_ref):
    out_ref[...] = (
        jnp.sum(x_ref[...] * w_ref[...], axis=1, keepdims=True) + b_ref[0, 0]
    )


def _matvec(X, Wt, b2):
    N, D = X.shape
    BLK = 4096
    return pl.pallas_call(
        _mv_body,
        grid=(N // BLK,),
        in_specs=[
            pl.BlockSpec((BLK, D), lambda i: (i, 0)),
            pl.BlockSpec((1, D), lambda i: (0, 0)),
            pl.BlockSpec(memory_space=pltpu.SMEM),
        ],
        out_specs=pl.BlockSpec((BLK, 1), lambda i: (i, 0)),
        out_shape=jax.ShapeDtypeStruct((N, 1), jnp.float32),
    )(X, Wt, b2)


def _sc_gather(table, bags2):
    nw, CHUNK = bags2.shape
    mesh = plsc.VectorSubcoreMesh(core_axis_name="c", subcore_axis_name="s")

    @functools.partial(
        pl.kernel,
        out_type=jax.ShapeDtypeStruct((nw, CHUNK), jnp.float32),
        mesh=mesh,
        scratch_types=[
            pltpu.VMEM((CHUNK,), jnp.int32),
            pltpu.VMEM((CHUNK,), jnp.float32),
            pltpu.SemaphoreType.DMA,
        ],
    )
    def k(table_hbm, bags_hbm, out_hbm, idx_v, vals_v, sem):
        wid = lax.axis_index("s") * 2 + lax.axis_index("c")
        pltpu.sync_copy(bags_hbm.at[wid], idx_v)
        pltpu.async_copy(table_hbm.at[idx_v], vals_v, sem).wait()
        pltpu.sync_copy(vals_v, out_hbm.at[wid])

    return k(table, bags2)


def _fin_body(g_ref, m_ref, out_ref):
    g = g_ref[...]
    msk = m_ref[...]
    g = jnp.where(msk != 0.0, -jnp.inf, g)
    m = jnp.max(g, axis=1, keepdims=True)
    sp = jnp.maximum(m, 0.0) + jnp.log(1.0 + jnp.exp(-jnp.abs(m)))
    out_ref[...] = jnp.concatenate([-sp, m - sp], axis=1)


def _finalize(g, maskf):
    B, L = g.shape
    BLK = 512
    return pl.pallas_call(
        _fin_body,
        grid=(B // BLK,),
        in_specs=[
            pl.BlockSpec((BLK, L), lambda i: (i, 0)),
            pl.BlockSpec((BLK, L), lambda i: (i, 0)),
        ],
        out_specs=pl.BlockSpec((BLK, 2), lambda i: (i, 0)),
        out_shape=jax.ShapeDtypeStruct((B, 2), jnp.float32),
    )(g, maskf)


def kernel(X, bags, padding_mask, W, b):
    N, D = X.shape
    B, L = bags.shape
    logits = _matvec(X, W.reshape(1, D), b.reshape(1, 1))
    table = logits.reshape(N)
    bags2 = bags.astype(jnp.int32).reshape(_NW, (B * L) // _NW)
    gathered = _sc_gather(table, bags2)
    return _finalize(gathered.reshape(B, L), padding_mask.astype(jnp.float32))


# speedup vs baseline: 25.2565x; 1.0216x over previous
import functools

import jax
import jax.numpy as jnp
from jax import lax
from jax.experimental import pallas as pl
from jax.experimental.pallas import tpu as pltpu
from jax.experimental.pallas import tpu_sc as plsc

_NW = 32


def _mv_body(x_ref, w_ref, b_ref, out_ref):
    out_ref[...] = (
        jnp.sum(x_ref[...] * w_ref[...], axis=1, keepdims=True) + b_ref[0, 0]
    )


def _matvec(X, Wt, b2):
    N, D = X.shape
    BLK = 8192
    return pl.pallas_call(
        _mv_body,
        grid=(N // BLK,),
        in_specs=[
            pl.BlockSpec((BLK, D), lambda i: (i, 0)),
            pl.BlockSpec((1, D), lambda i: (0, 0)),
            pl.BlockSpec(memory_space=pltpu.SMEM),
        ],
        out_specs=pl.BlockSpec((BLK, 1), lambda i: (i, 0)),
        out_shape=jax.ShapeDtypeStruct((N, 1), jnp.float32),
    )(X, Wt, b2)


def _sc_gather(table, bags2):
    nw, CHUNK = bags2.shape
    mesh = plsc.VectorSubcoreMesh(core_axis_name="c", subcore_axis_name="s")

    @functools.partial(
        pl.kernel,
        out_type=jax.ShapeDtypeStruct((nw, CHUNK), jnp.float32),
        mesh=mesh,
        scratch_types=[
            pltpu.VMEM((CHUNK,), jnp.int32),
            pltpu.VMEM((CHUNK,), jnp.float32),
            pltpu.SemaphoreType.DMA,
        ],
    )
    def k(table_hbm, bags_hbm, out_hbm, idx_v, vals_v, sem):
        wid = lax.axis_index("s") * 2 + lax.axis_index("c")
        pltpu.sync_copy(bags_hbm.at[wid], idx_v)
        pltpu.async_copy(table_hbm.at[idx_v], vals_v, sem).wait()
        pltpu.sync_copy(vals_v, out_hbm.at[wid])

    return k(table, bags2)


def _fin_body(g_ref, m_ref, out_ref):
    g = g_ref[...]
    msk = m_ref[...]
    g = jnp.where(msk != 0.0, -jnp.inf, g)
    m = jnp.max(g, axis=1, keepdims=True)
    sp = jnp.maximum(m, 0.0) + jnp.log(1.0 + jnp.exp(-jnp.abs(m)))
    out_ref[...] = jnp.concatenate([-sp, m - sp], axis=1)


def _finalize(g, maskf):
    B, L = g.shape
    BLK = 512
    return pl.pallas_call(
        _fin_body,
        grid=(B // BLK,),
        in_specs=[
            pl.BlockSpec((BLK, L), lambda i: (i, 0)),
            pl.BlockSpec((BLK, L), lambda i: (i, 0)),
        ],
        out_specs=pl.BlockSpec((BLK, 2), lambda i: (i, 0)),
        out_shape=jax.ShapeDtypeStruct((B, 2), jnp.float32),
    )(g, maskf)


def kernel(X, bags, padding_mask, W, b):
    N, D = X.shape
    B, L = bags.shape
    logits = _matvec(X, W.reshape(1, D), b.reshape(1, 1))
    table = logits.reshape(N)
    bags2 = bags.astype(jnp.int32).reshape(_NW, (B * L) // _NW)
    gathered = _sc_gather(table, bags2)
    return _finalize(gathered.reshape(B, L), padding_mask.astype(jnp.float32))


# speedup vs baseline: 25.3320x; 1.0030x over previous
import functools

import jax
import jax.numpy as jnp
from jax import lax
from jax.experimental import pallas as pl
from jax.experimental.pallas import tpu as pltpu
from jax.experimental.pallas import tpu_sc as plsc

_NW = 32


def _mv_body(x_ref, w_ref, b_ref, out_ref):
    out_ref[...] = (
        jnp.sum(x_ref[...] * w_ref[...], axis=1, keepdims=True) + b_ref[0, 0]
    )


def _matvec(X, Wt, b2):
    N, D = X.shape
    BLK = 16384
    return pl.pallas_call(
        _mv_body,
        grid=(N // BLK,),
        in_specs=[
            pl.BlockSpec((BLK, D), lambda i: (i, 0)),
            pl.BlockSpec((1, D), lambda i: (0, 0)),
            pl.BlockSpec(memory_space=pltpu.SMEM),
        ],
        out_specs=pl.BlockSpec((BLK, 1), lambda i: (i, 0)),
        out_shape=jax.ShapeDtypeStruct((N, 1), jnp.float32),
    )(X, Wt, b2)


def _sc_gather(table, bags2):
    nw, CHUNK = bags2.shape
    mesh = plsc.VectorSubcoreMesh(core_axis_name="c", subcore_axis_name="s")

    @functools.partial(
        pl.kernel,
        out_type=jax.ShapeDtypeStruct((nw, CHUNK), jnp.float32),
        mesh=mesh,
        scratch_types=[
            pltpu.VMEM((CHUNK,), jnp.int32),
            pltpu.VMEM((CHUNK,), jnp.float32),
            pltpu.SemaphoreType.DMA,
        ],
    )
    def k(table_hbm, bags_hbm, out_hbm, idx_v, vals_v, sem):
        wid = lax.axis_index("s") * 2 + lax.axis_index("c")
        pltpu.sync_copy(bags_hbm.at[wid], idx_v)
        pltpu.async_copy(table_hbm.at[idx_v], vals_v, sem).wait()
        pltpu.sync_copy(vals_v, out_hbm.at[wid])

    return k(table, bags2)


def _fin_body(g_ref, m_ref, out_ref):
    g = g_ref[...]
    msk = m_ref[...]
    g = jnp.where(msk != 0.0, -jnp.inf, g)
    m = jnp.max(g, axis=1, keepdims=True)
    sp = jnp.maximum(m, 0.0) + jnp.log(1.0 + jnp.exp(-jnp.abs(m)))
    out_ref[...] = jnp.concatenate([-sp, m - sp], axis=1)


def _finalize(g, maskf):
    B, L = g.shape
    BLK = 512
    return pl.pallas_call(
        _fin_body,
        grid=(B // BLK,),
        in_specs=[
            pl.BlockSpec((BLK, L), lambda i: (i, 0)),
            pl.BlockSpec((BLK, L), lambda i: (i, 0)),
        ],
        out_specs=pl.BlockSpec((BLK, 2), lambda i: (i, 0)),
        out_shape=jax.ShapeDtypeStruct((B, 2), jnp.float32),
    )(g, maskf)


def kernel(X, bags, padding_mask, W, b):
    N, D = X.shape
    B, L = bags.shape
    logits = _matvec(X, W.reshape(1, D), b.reshape(1, 1))
    table = logits.reshape(N)
    bags2 = bags.astype(jnp.int32).reshape(_NW, (B * L) // _NW)
    gathered = _sc_gather(table, bags2)
    return _finalize(gathered.reshape(B, L), padding_mask.astype(jnp.float32))


# speedup vs baseline: 25.3403x; 1.0003x over previous
import functools

import jax
import jax.numpy as jnp
from jax import lax
from jax.experimental import pallas as pl
from jax.experimental.pallas import tpu as pltpu
from jax.experimental.pallas import tpu_sc as plsc

_NW = 32


def _mv_body(x_ref, w_ref, b_ref, out_ref):
    out_ref[...] = (
        jnp.sum(x_ref[...] * w_ref[...], axis=1, keepdims=True) + b_ref[0, 0]
    )


def _matvec(X, Wt, b2):
    N, D = X.shape
    BLK = 16384
    return pl.pallas_call(
        _mv_body,
        grid=(N // BLK,),
        in_specs=[
            pl.BlockSpec((BLK, D), lambda i: (i, 0)),
            pl.BlockSpec((1, D), lambda i: (0, 0)),
            pl.BlockSpec(memory_space=pltpu.SMEM),
        ],
        out_specs=pl.BlockSpec((BLK, 1), lambda i: (i, 0)),
        out_shape=jax.ShapeDtypeStruct((N, 1), jnp.float32),
    )(X, Wt, b2)


def _sc_gather_max(table, bags2, mask2, L):
    nw, CHUNK = bags2.shape
    bags_per_tile = CHUNK // L
    vregs_per_bag = L // 16
    mesh = plsc.VectorSubcoreMesh(core_axis_name="c", subcore_axis_name="s")

    @functools.partial(
        pl.kernel,
        out_type=jax.ShapeDtypeStruct((nw, bags_per_tile), jnp.float32),
        mesh=mesh,
        scratch_types=[
            pltpu.VMEM((CHUNK,), jnp.int32),
            pltpu.VMEM((CHUNK,), jnp.int32),
            pltpu.VMEM((CHUNK,), jnp.float32),
            pltpu.VMEM((bags_per_tile,), jnp.float32),
            pltpu.SemaphoreType.DMA,
        ],
    )
    def k(table_hbm, bags_hbm, mask_hbm, out_hbm, idx_v, msk_v, vals_v, res_v, sem):
        wid = lax.axis_index("s") * 2 + lax.axis_index("c")
        pltpu.sync_copy(bags_hbm.at[wid], idx_v)
        cp = pltpu.async_copy(table_hbm.at[idx_v], vals_v, sem)
        pltpu.sync_copy(mask_hbm.at[wid], msk_v)
        cp.wait()
        neg = jnp.full((16,), -jnp.inf, dtype=jnp.float32)
        lane = lax.iota(jnp.int32, 16)

        dnums = lax.GatherDimensionNumbers(
            offset_dims=(), collapsed_slice_dims=(0,), start_index_map=(0,)
        )

        def vmax16(a):
            for s in (1, 2, 4, 8):
                perm = jnp.bitwise_xor(lane, s)
                shuf = lax.gather(
                    a, perm[:, None], dnums, (1,),
                    mode=lax.GatherScatterMode.PROMISE_IN_BOUNDS,
                )
                a = jnp.maximum(a, shuf)
            return a

        def group_body(g, carry):
            res = neg
            for k in range(16):
                off0 = (g * 16 + k) * L
                acc = neg
                for i in range(vregs_per_bag):
                    off = off0 + i * 16
                    v = vals_v[pl.ds(off, 16)]
                    m = msk_v[pl.ds(off, 16)]
                    acc = jnp.maximum(acc, jnp.where(m != 0, -jnp.inf, v))
                res = jnp.where(lane == k, vmax16(acc), res)
            res_v[pl.ds(g * 16, 16)] = res
            return carry

        lax.fori_loop(0, bags_per_tile // 16, group_body, 0)
        pltpu.sync_copy(res_v, out_hbm.at[wid])

    return k(table, bags2, mask2)


def _fin_body(m_ref, o0_ref, o1_ref):
    m = m_ref[...]
    sp = jnp.maximum(m, 0.0) + jnp.log(1.0 + jnp.exp(-jnp.abs(m)))
    o0_ref[...] = -sp
    o1_ref[...] = m - sp


def _finalize(m2):
    R, C = m2.shape
    out = jax.ShapeDtypeStruct((R, C), jnp.float32)
    return pl.pallas_call(
        _fin_body,
        out_shape=(out, out),
    )(m2)


def kernel(X, bags, padding_mask, W, b):
    N, D = X.shape
    B, L = bags.shape
    logits = _matvec(X, W.reshape(1, D), b.reshape(1, 1))
    table = logits.reshape(N)
    bags2 = bags.astype(jnp.int32).reshape(_NW, (B * L) // _NW)
    mask2 = padding_mask.astype(jnp.int32).reshape(_NW, (B * L) // _NW)
    m = _sc_gather_max(table, bags2, mask2, L)
    o0, o1 = _finalize(m)
    return jnp.stack([o0.reshape(B), o1.reshape(B)], axis=-1)
